# baseline (speedup 1.0000x reference)
import jax
import jax.numpy as jnp
from jax import lax
from jax.experimental import pallas as pl
from jax.experimental.pallas import tpu as pltpu
from jax.experimental.pallas import tpu_sc as plsc

NC = 2
NS = 16
NW = NC * NS
SEQ = 8192
TOK = SEQ // NW
D = 50
V = 1000000
BLK = 65536
NBLK = (V + BLK - 1) // BLK
VP = NBLK * BLK
HSLC = VP // NS
ZB = HSLC // 8


def _sc_hist(x_hbm, hist_hbm, idx_v, ones_v, zero_v, hist_s, sem, zsem):
    cid = lax.axis_index("c")
    sid = lax.axis_index("s")
    wid = sid * NC + cid

    def zfill(i, _):
        zero_v[pl.ds(i * 16, 16)] = jnp.zeros((16,), jnp.float32)
        return 0

    for j in range(TOK // 128):
        pltpu.async_copy(x_hbm.at[pl.ds(wid * TOK + j * 128, 128)], idx_v.at[j], sem)

    lax.fori_loop(0, ZB // 16, zfill, 0)

    def ofill(i, _):
        ones_v[pl.ds(i * 16, 16)] = jnp.ones((16,), jnp.float32)
        return 0

    lax.fori_loop(0, 128 // 16, ofill, 0)
    for r in range(8):
        pltpu.async_copy(zero_v, hist_s.at[pl.ds(sid * HSLC + r * ZB, ZB)], zsem)
    for j in range(TOK // 128):
        pltpu.make_async_copy(x_hbm.at[pl.ds(wid * TOK + j * 128, 128)], idx_v.at[j], sem).wait()
    for r in range(8):
        pltpu.make_async_copy(zero_v, hist_s.at[pl.ds(sid * HSLC + r * ZB, ZB)], zsem).wait()
    plsc.subcore_barrier()
    for j in range(TOK // 128):
        pltpu.sync_copy(ones_v, hist_s.at[idx_v.at[j]], add=True)
    plsc.subcore_barrier()

    pltpu.sync_copy(
        hist_s.at[pl.ds(sid * HSLC, HSLC)],
        hist_hbm.at[pl.ds(cid * VP + sid * HSLC, HSLC)],
    )


@jax.jit
def _stage1(x1d):
    mesh = plsc.VectorSubcoreMesh(core_axis_name="c", subcore_axis_name="s")
    f = pl.kernel(
        _sc_hist,
        out_type=jax.ShapeDtypeStruct((NC * VP,), jnp.float32),
        mesh=mesh,
        scratch_types=[
            pltpu.VMEM((TOK // 128, 128), jnp.int32),
            pltpu.VMEM((128,), jnp.float32),
            pltpu.VMEM((ZB,), jnp.float32),
            pltpu.VMEM_SHARED((VP,), jnp.float32),
            pltpu.SemaphoreType.DMA,
            pltpu.SemaphoreType.DMA,
        ],
    )
    return f(x1d)


def _tc_contract(embt_ref, hist0_ref, hist1_ref, w8_ref, b8_ref, out_ref,
                 s_ref, c_ref):
    i = pl.program_id(0)

    @pl.when(i == 0)
    def _():
        s_ref[...] = jnp.zeros_like(s_ref)
        c_ref[...] = jnp.zeros_like(c_ref)

    h = (hist0_ref[...] + hist1_ref[...]).reshape(1, BLK)

    def accumulate(e):
        eh = e * h
        mh = jnp.where(e != 0.0, h, 0.0)
        s_ref[...] += jnp.sum(eh, axis=1, keepdims=True)
        c_ref[...] += jnp.sum(mh, axis=1, keepdims=True)

    @pl.when(i < NBLK - 1)
    def _():
        accumulate(embt_ref[...])

    @pl.when(i == NBLK - 1)
    def _():
        col = lax.broadcasted_iota(jnp.int32, (D, BLK), 1)
        accumulate(jnp.where(col < V - (NBLK - 1) * BLK, embt_ref[...], 0.0))
        s_col = s_ref[...]
        ms_col = c_ref[...]
        sw = lax.dot_general(s_col, w8_ref[...],
                             (((0,), (1,)), ((), ())),
                             preferred_element_type=jnp.float32)
        denom = ms_col + jnp.where(ms_col == 0.0, 1e-10, 0.0)
        out_ref[...] = (1.0 / denom) * sw + b8_ref[...]


@jax.jit
def _stage2(embt, hist, w8, b8):
    return pl.pallas_call(
        _tc_contract,
        grid=(NBLK,),
        in_specs=[
            pl.BlockSpec((D, BLK), lambda i: (0, i)),
            pl.BlockSpec((BLK,), lambda i: (i,)),
            pl.BlockSpec((BLK,), lambda i: (NBLK + i,)),
            pl.BlockSpec((8, D), lambda i: (0, 0)),
            pl.BlockSpec((1, 8), lambda i: (0, 0)),
        ],
        out_specs=pl.BlockSpec((D, 8), lambda i: (0, 0)),
        out_shape=jax.ShapeDtypeStruct((D, 8), jnp.float32),
        scratch_shapes=[
            pltpu.VMEM((D, 1), jnp.float32),
            pltpu.VMEM((D, 1), jnp.float32),
        ],
    )(embt, hist, hist, w8, b8)


def kernel(x, emb, W, b):
    hist = _stage1(x.reshape(SEQ))
    hist2 = _stage1(x.reshape(SEQ) * 0 + 1)
    embt = jnp.swapaxes(emb, 0, 1)
    w8 = jnp.zeros((8, D), jnp.float32).at[:3].set(W)
    b8 = jnp.zeros((1, 8), jnp.float32).at[0, :3].set(b)
    y = _stage2(embt, hist, w8, b8)
    return (y[:, :3] + 0.0 * hist2[0])[None]


# speedup vs baseline: 1.1662x; 1.1662x over previous
import jax
import jax.numpy as jnp
from jax import lax
from jax.experimental import pallas as pl
from jax.experimental.pallas import tpu as pltpu
from jax.experimental.pallas import tpu_sc as plsc

NC = 2
NS = 16
NW = NC * NS
SEQ = 8192
TOK = SEQ // NW
D = 50
V = 1000000
BLK = 98304
NBLK = (V + BLK - 1) // BLK
VP = NBLK * BLK
HSLC = VP // NS
ZB = HSLC // 8


def _sc_hist(x_hbm, hist_hbm, idx_v, ones_v, zero_v, hist_s, sem, zsem):
    cid = lax.axis_index("c")
    sid = lax.axis_index("s")
    wid = sid * NC + cid

    def zfill(i, _):
        zero_v[pl.ds(i * 16, 16)] = jnp.zeros((16,), jnp.float32)
        return 0

    for j in range(TOK // 128):
        pltpu.async_copy(x_hbm.at[pl.ds(wid * TOK + j * 128, 128)], idx_v.at[j], sem)

    lax.fori_loop(0, ZB // 16, zfill, 0)

    def ofill(i, _):
        ones_v[pl.ds(i * 16, 16)] = jnp.ones((16,), jnp.float32)
        return 0

    lax.fori_loop(0, 128 // 16, ofill, 0)
    for r in range(8):
        pltpu.async_copy(zero_v, hist_s.at[pl.ds(sid * HSLC + r * ZB, ZB)], zsem)
    for j in range(TOK // 128):
        pltpu.make_async_copy(x_hbm.at[pl.ds(wid * TOK + j * 128, 128)], idx_v.at[j], sem).wait()
    for r in range(8):
        pltpu.make_async_copy(zero_v, hist_s.at[pl.ds(sid * HSLC + r * ZB, ZB)], zsem).wait()
    plsc.subcore_barrier()
    for j in range(TOK // 128):
        pltpu.sync_copy(ones_v, hist_s.at[idx_v.at[j]], add=True)
    plsc.subcore_barrier()

    pltpu.sync_copy(
        hist_s.at[pl.ds(sid * HSLC, HSLC)],
        hist_hbm.at[pl.ds(cid * VP + sid * HSLC, HSLC)],
    )


@jax.jit
def _stage1(x1d):
    mesh = plsc.VectorSubcoreMesh(core_axis_name="c", subcore_axis_name="s")
    f = pl.kernel(
        _sc_hist,
        out_type=jax.ShapeDtypeStruct((NC * VP,), jnp.float32),
        mesh=mesh,
        scratch_types=[
            pltpu.VMEM((TOK // 128, 128), jnp.int32),
            pltpu.VMEM((128,), jnp.float32),
            pltpu.VMEM((ZB,), jnp.float32),
            pltpu.VMEM_SHARED((VP,), jnp.float32),
            pltpu.SemaphoreType.DMA,
            pltpu.SemaphoreType.DMA,
        ],
    )
    return f(x1d)


def _tc_contract(embt_ref, hist0_ref, hist1_ref, w8_ref, b8_ref, out_ref,
                 s_ref, c_ref):
    i = pl.program_id(0)

    @pl.when(i == 0)
    def _():
        s_ref[...] = jnp.zeros_like(s_ref)
        c_ref[...] = jnp.zeros_like(c_ref)

    h = (hist0_ref[...] + hist1_ref[...]).reshape(1, BLK)

    def accumulate(e):
        eh = e * h
        mh = jnp.where(e != 0.0, h, 0.0)
        s_ref[...] += jnp.sum(eh, axis=1, keepdims=True)
        c_ref[...] += jnp.sum(mh, axis=1, keepdims=True)

    @pl.when(i < NBLK - 1)
    def _():
        accumulate(embt_ref[...])

    @pl.when(i == NBLK - 1)
    def _():
        col = lax.broadcasted_iota(jnp.int32, (D, BLK), 1)
        accumulate(jnp.where(col < V - (NBLK - 1) * BLK, embt_ref[...], 0.0))
        s_col = s_ref[...]
        ms_col = c_ref[...]
        sw = lax.dot_general(s_col, w8_ref[...],
                             (((0,), (1,)), ((), ())),
                             preferred_element_type=jnp.float32)
        denom = ms_col + jnp.where(ms_col == 0.0, 1e-10, 0.0)
        out_ref[...] = (1.0 / denom) * sw + b8_ref[...]


@jax.jit
def _stage2(embt, hist, w8, b8):
    return pl.pallas_call(
        _tc_contract,
        grid=(NBLK,),
        in_specs=[
            pl.BlockSpec((D, BLK), lambda i: (0, i)),
            pl.BlockSpec((BLK,), lambda i: (i,)),
            pl.BlockSpec((BLK,), lambda i: (NBLK + i,)),
            pl.BlockSpec((8, D), lambda i: (0, 0)),
            pl.BlockSpec((1, 8), lambda i: (0, 0)),
        ],
        out_specs=pl.BlockSpec((D, 8), lambda i: (0, 0)),
        out_shape=jax.ShapeDtypeStruct((D, 8), jnp.float32),
        scratch_shapes=[
            pltpu.VMEM((D, 1), jnp.float32),
            pltpu.VMEM((D, 1), jnp.float32),
        ],
    )(embt, hist, hist, w8, b8)


def kernel(x, emb, W, b):
    hist = _stage1(x.reshape(SEQ))
    embt = jnp.swapaxes(emb, 0, 1)
    w8 = jnp.zeros((8, D), jnp.float32).at[:3].set(W)
    b8 = jnp.zeros((1, 8), jnp.float32).at[0, :3].set(b)
    y = _stage2(embt, hist, w8, b8)
    return y[:, :3][None]
